# baseline (speedup 1.0000x reference)
import functools

import jax
import jax.numpy as jnp
from jax import lax
from jax.experimental import pallas as pl
from jax.experimental.pallas import tpu as pltpu
from jax.experimental.pallas import tpu_sc as plsc

_ROWS = 8192
_COLS = 1024
_NC = 2
_NS = 16
_NW = _NC * _NS
_LANES = 16
_GROUPS = 4
_BLK = _GROUPS * _LANES
_ROWS_PER_W = _ROWS // _NW
_NBLK = _ROWS_PER_W // _BLK


def _sc_body(x_hbm, o_hbm, buf):
    wid = lax.axis_index("c") * _NS + lax.axis_index("s")
    row0 = wid * _ROWS_PER_W

    lane = lax.broadcasted_iota(jnp.int32, (_LANES,), 0)
    ridx = [lane + g * _LANES for g in range(_GROUPS)]

    def block_body(b, _):
        base = row0 + b * _BLK
        pltpu.sync_copy(x_hbm.at[pl.ds(base, _BLK), :], buf)

        zero = jnp.zeros((_LANES,), jnp.int32)
        accs = [plsc.load_gather(buf, [ridx[g], zero]) for g in range(_GROUPS)]

        def col_body(j, accs):
            cj = jnp.full((_LANES,), j, jnp.int32)
            out = []
            for g in range(_GROUPS):
                v = plsc.load_gather(buf, [ridx[g], cj])
                plsc.store_scatter(buf, [ridx[g], cj], accs[g])
                out.append(accs[g] + v)
            return tuple(out)

        lax.fori_loop(1, _COLS, col_body, tuple(accs), unroll=2)
        pltpu.sync_copy(buf, o_hbm.at[pl.ds(base, _BLK), :])
        return 0

    lax.fori_loop(0, _NBLK, block_body, 0)


def kernel(x):
    mesh = plsc.VectorSubcoreMesh(
        core_axis_name="c", subcore_axis_name="s",
        num_cores=_NC, num_subcores=_NS,
    )
    f = functools.partial(
        pl.kernel,
        out_type=jax.ShapeDtypeStruct((_ROWS, _COLS), jnp.float32),
        mesh=mesh,
        scratch_types=[pltpu.VMEM((_BLK, _COLS), jnp.float32)],
        compiler_params=pltpu.CompilerParams(
            use_tc_tiling_on_sc=False, needs_layout_passes=False),
    )(_sc_body)
    return f(x)
